# baseline (speedup 1.0000x reference)
import functools

import jax
import jax.numpy as jnp
from jax.experimental import pallas as pl
from jax.experimental.pallas import tpu as pltpu

_NBLK = 1000


def _two_mat_body(x1_ref, x2_ref, w1_ref, w2_ref, b_ref, g_ref, o_ref, *, act):
    x2 = x2_ref[...] + g_ref[...]
    y = (jnp.dot(x1_ref[...], w1_ref[...], preferred_element_type=jnp.float32)
         + jnp.dot(x2, w2_ref[...], preferred_element_type=jnp.float32)
         + b_ref[...])
    if act:
        y = jnp.maximum(y, 0.0)
    o_ref[...] = y


def _two_mat(x1, x2, w1, w2, b, g, act):
    n, c = x1.shape
    co = w1.shape[1]
    grid = (n // _NBLK,)
    return pl.pallas_call(
        functools.partial(_two_mat_body, act=act),
        grid=grid,
        in_specs=[
            pl.BlockSpec((_NBLK, c), lambda i: (i, 0)),
            pl.BlockSpec((_NBLK, c), lambda i: (i, 0)),
            pl.BlockSpec((c, co), lambda i: (0, 0)),
            pl.BlockSpec((c, co), lambda i: (0, 0)),
            pl.BlockSpec((1, co), lambda i: (0, 0)),
            pl.BlockSpec((1, c), lambda i: (0, 0)),
        ],
        out_specs=pl.BlockSpec((_NBLK, co), lambda i: (i, 0)),
        out_shape=jax.ShapeDtypeStruct((n, co), jnp.float32),
    )(x1, x2, w1, w2, b, g)


def _step_dense_body(xc_ref, wl_ref, wr_ref, wcx_ref, pc_ref, cc_ref, bt_ref,
                     xl_ref, ar_ref, xvc_ref):
    xc = xc_ref[...]
    nb = bt_ref.shape[0]
    bcast = jax.lax.broadcasted_iota(jnp.int32, (nb, pc_ref.shape[0]), 1)
    onehot = (bt_ref[...] == bcast).astype(jnp.float32)
    xl_ref[...] = jnp.dot(xc, wl_ref[...], preferred_element_type=jnp.float32)
    pcn = jnp.dot(onehot, pc_ref[...], preferred_element_type=jnp.float32)
    ccn = jnp.dot(onehot, cc_ref[...], preferred_element_type=jnp.float32)
    xr = jnp.dot(xc, wr_ref[...], preferred_element_type=jnp.float32)
    ar_ref[...] = pcn * xr
    xcx = jnp.dot(xc, wcx_ref[...], preferred_element_type=jnp.float32)
    xvc_ref[...] = xcx * ccn


def _step_dense(x_ctx, wl, wr, wcx, pc, cc, batch2d):
    n, c = x_ctx.shape
    nb = pc.shape[0]
    grid = (n // _NBLK,)
    shp = jax.ShapeDtypeStruct((n, c), jnp.float32)
    return pl.pallas_call(
        _step_dense_body,
        grid=grid,
        in_specs=[
            pl.BlockSpec((_NBLK, c), lambda i: (i, 0)),
            pl.BlockSpec((c, c), lambda i: (0, 0)),
            pl.BlockSpec((c, c), lambda i: (0, 0)),
            pl.BlockSpec((c, c), lambda i: (0, 0)),
            pl.BlockSpec((nb, c), lambda i: (0, 0)),
            pl.BlockSpec((nb, c), lambda i: (0, 0)),
            pl.BlockSpec((_NBLK, 1), lambda i: (i, 0)),
        ],
        out_specs=[
            pl.BlockSpec((_NBLK, c), lambda i: (i, 0)),
            pl.BlockSpec((_NBLK, c), lambda i: (i, 0)),
            pl.BlockSpec((_NBLK, c), lambda i: (i, 0)),
        ],
        out_shape=[shp, shp, shp],
    )(x_ctx, wl, wr, wcx, pc, cc, batch2d)


def _cmd_body(q1_ref, q2w_ref, q2b_ref, w_refs, c2lw_ref, c2lb_ref,
              wpc_ref, wcc_ref, pc_ref, cc_ref, *, ni):
    cmd_t = (jnp.dot(jnp.maximum(q1_ref[...], 0.0), q2w_ref[...],
                     preferred_element_type=jnp.float32) + q2b_ref[...])
    u = cmd_t * c2lw_ref[...]
    logits = []
    for i in range(ni):
        wi = w_refs[i][...]
        logits.append(jnp.sum(u * wi, axis=-1, keepdims=True))
    lg = jnp.concatenate(logits, axis=1) + c2lb_ref[0, 0]
    lg = lg - jnp.max(lg, axis=1, keepdims=True)
    e = jnp.exp(lg)
    att = e / jnp.sum(e, axis=1, keepdims=True)
    cmd = att[:, 0:1] * w_refs[0][...]
    for i in range(1, ni):
        cmd = cmd + att[:, i:i + 1] * w_refs[i][...]
    pc_ref[...] = jnp.dot(cmd, wpc_ref[...], preferred_element_type=jnp.float32)
    cc_ref[...] = jnp.dot(cmd, wcc_ref[...], preferred_element_type=jnp.float32)


def _cmd_step(q1, q2w, q2b, words, c2l_w, c2l_b, wpc, wcc):
    b, c = q1.shape
    ni = words.shape[1]
    word_slices = [words[:, i, :] for i in range(ni)]
    full = lambda s: pl.BlockSpec(s, lambda: (0, 0))
    body = functools.partial(_cmd_body, ni=ni)

    def wrapper(q1_ref, q2w_ref, q2b_ref, *rest):
        w_refs = rest[:ni]
        c2lw_ref, c2lb_ref, wpc_ref, wcc_ref, pc_ref, cc_ref = rest[ni:]
        body(q1_ref, q2w_ref, q2b_ref, w_refs, c2lw_ref, c2lb_ref,
             wpc_ref, wcc_ref, pc_ref, cc_ref)

    return pl.pallas_call(
        wrapper,
        in_specs=[full((b, c)), full((c, c)), full((1, c))]
                 + [full((b, c)) for _ in range(ni)]
                 + [full((1, c)), full((1, 1)), full((c, c)), full((c, c))],
        out_specs=[full((b, c)), full((b, c))],
        out_shape=[jax.ShapeDtypeStruct((b, c), jnp.float32)] * 2,
    )(q1, q2w, q2b.reshape(1, c), *word_slices, c2l_w.reshape(1, c),
      c2l_b.reshape(1, 1), wpc, wcc)


def _edge_body(sd_ref, xl_ref, ar_ref, xvc_ref, agg_ref,
               amax_scr, asum_scr, *, n_nodes, n_edges):
    def unpack(e):
        p = sd_ref[e]
        return jax.lax.shift_right_logical(p, 14), jax.lax.bitwise_and(p, 16383)

    amax_scr[...] = jnp.full(amax_scr.shape, -jnp.inf, jnp.float32)
    asum_scr[...] = jnp.zeros(asum_scr.shape, jnp.float32)
    agg_ref[...] = jnp.zeros(agg_ref.shape, jnp.float32)

    def logit(s, d):
        a = jnp.sum(xl_ref[pl.ds(s, 1), :] * ar_ref[pl.ds(d, 1), :])
        return jnp.where(a >= 0.0, a, 0.2 * a)

    def body1(e, carry):
        s, d = unpack(e)
        a = logit(s, d)
        m = amax_scr[pl.ds(d, 1), :]
        amax_scr[pl.ds(d, 1), :] = jnp.maximum(m, a.reshape(1, 1))
        return carry

    jax.lax.fori_loop(0, n_edges, body1, 0)

    def body2(e, carry):
        s, d = unpack(e)
        a = logit(s, d)
        v = jnp.exp(a.reshape(1, 1) - amax_scr[pl.ds(d, 1), :])
        asum_scr[pl.ds(d, 1), :] = asum_scr[pl.ds(d, 1), :] + v
        return carry

    jax.lax.fori_loop(0, n_edges, body2, 0)

    def body3(e, carry):
        s, d = unpack(e)
        a = logit(s, d)
        v = jnp.exp(a.reshape(1, 1) - amax_scr[pl.ds(d, 1), :])
        alpha = v / (asum_scr[pl.ds(d, 1), :] + 1e-16)
        agg_ref[pl.ds(d, 1), :] = (agg_ref[pl.ds(d, 1), :]
                                   + xvc_ref[pl.ds(s, 1), :] * alpha)
        return carry

    jax.lax.fori_loop(0, n_edges, body3, 0)


def _edge_step(sd_packed, xl, ar, xvc):
    n, c = xl.shape
    n_edges = sd_packed.shape[0]
    grid_spec = pltpu.PrefetchScalarGridSpec(
        num_scalar_prefetch=1,
        grid=(1,),
        in_specs=[
            pl.BlockSpec((n, c), lambda i, sd: (0, 0)),
            pl.BlockSpec((n, c), lambda i, sd: (0, 0)),
            pl.BlockSpec((n, c), lambda i, sd: (0, 0)),
        ],
        out_specs=pl.BlockSpec((n, c), lambda i, sd: (0, 0)),
        scratch_shapes=[
            pltpu.VMEM((n, 1), jnp.float32),
            pltpu.VMEM((n, 1), jnp.float32),
        ],
    )
    return pl.pallas_call(
        functools.partial(_edge_body, n_nodes=n, n_edges=n_edges),
        grid_spec=grid_spec,
        out_shape=jax.ShapeDtypeStruct((n, c), jnp.float32),
    )(sd_packed, xl, ar, xvc)


def kernel(x, edge_index, q, words, batch, init_W, init_b, q1_W, q1_b, q2_W,
           q2_b, c2l_W, c2l_b, ploc_W, ploc_b, pctx_W, pctx_b, out_W, out_b,
           fin_W, fin_b, Wl, Wr, Wcx, Wpc, Wcc, gat_bias):
    n, c = x.shape
    nt = q2_W.shape[0]
    loop = jnp.arange(n, dtype=edge_index.dtype)
    src = jnp.concatenate([edge_index[0], loop])
    dst = jnp.concatenate([edge_index[1], loop])
    sd_packed = (src << 14) | dst
    batch2d = batch.reshape(n, 1)
    zc = jnp.zeros((1, c), jnp.float32)
    zw = jnp.zeros((c, c), jnp.float32)

    x_emb = _two_mat(x, x, init_W, zw, init_b.reshape(1, c), zc, act=False)
    x_loc = _two_mat(x_emb, x_emb, ploc_W, zw, ploc_b.reshape(1, c), zc,
                     act=False)
    x_ctx = _two_mat(x_emb, x_emb, pctx_W, zw, pctx_b.reshape(1, c), zc,
                     act=False)
    q1 = q @ q1_W + q1_b

    ow1 = out_W[:c, :]
    ow2 = out_W[c:, :]
    for t in range(nt):
        pc, cc = _cmd_step(q1, q2_W[t], q2_b[t], words, c2l_W[:, 0], c2l_b,
                           Wpc[t], Wcc[t])
        xl, ar, xvc = _step_dense(x_ctx, Wl[t], Wr[t], Wcx[t], pc, cc, batch2d)
        agg = _edge_step(sd_packed, xl, ar, xvc)
        x_ctx = _two_mat(x_ctx, agg, ow1, ow2, out_b.reshape(1, c),
                         gat_bias[t].reshape(1, c), act=True)
    out = _two_mat(x_loc, x_ctx, fin_W[:c, :], fin_W[c:, :],
                   fin_b.reshape(1, c), zc, act=False)
    return out
